# baseline (device time: 272882 ns/iter reference)
import jax
import jax.numpy as jnp
from jax import lax
from jax.experimental import pallas as pl
from jax.experimental.pallas import tpu as pltpu

N_DEV = 4
N_HOP = N_DEV - 1
S = 2


def kernel(A, B):
    A = A.astype(jnp.bfloat16)
    B = B.astype(jnp.bfloat16)
    m_per, k = A.shape
    n = B.shape[1]
    m_half = m_per // 2
    m_q = m_half // S

    def body(a_ref, b_ref, dummy_ref, out_ref, cw_ref, ccw_ref, chunk_ref,
             send_cw, recv_cw, send_ccw, recv_ccw, copy_sems):
        del dummy_ref
        my_pos = lax.axis_index("i")
        left = (my_pos - 1) % N_DEV
        right = (my_pos + 1) % N_DEV

        barrier_sem = pltpu.get_barrier_semaphore()
        for nbr in (left, right):
            pl.semaphore_signal(
                barrier_sem, inc=1,
                device_id=(nbr,), device_id_type=pl.DeviceIdType.MESH,
            )
        pl.semaphore_wait(barrier_sem, 2)

        def rdma(h, q, cw):
            buf, qsend, qrecv, tgt, a_off = (
                (cw_ref, send_cw, recv_cw, right, 0) if cw
                else (ccw_ref, send_ccw, recv_ccw, left, m_half)
            )
            rows = pl.ds(q * m_q, m_q)
            src = (a_ref.at[pl.ds(a_off + q * m_q, m_q), :] if h == 0
                   else buf.at[h - 1, rows, :])
            return pltpu.make_async_remote_copy(
                src_ref=src, dst_ref=buf.at[h, rows, :],
                send_sem=qsend.at[h * S + q], recv_sem=qrecv.at[h * S + q],
                device_id=(tgt,), device_id_type=pl.DeviceIdType.MESH,
            )

        copies = [None, None]
        gemm_count = [0]

        def quarter_gemm(src_view, out_row):
            j = gemm_count[0] % 2
            gemm_count[0] += 1
            if copies[j] is not None:
                copies[j].wait()
            chunk_ref[j] = jnp.dot(
                src_view, b_ref[...], preferred_element_type=jnp.float32,
            ).astype(jnp.bfloat16)
            copy = pltpu.make_async_copy(
                chunk_ref.at[j], out_ref.at[pl.ds(out_row, m_q), :],
                copy_sems.at[j],
            )
            copy.start()
            copies[j] = copy

        sends = []
        for q in range(S):
            for cw in (True, False):
                r = rdma(0, q, cw)
                r.start()
                sends.append(r)

        for qq in range(2 * S):
            quarter_gemm(
                a_ref[pl.ds(qq * m_q, m_q), :],
                my_pos * m_per + qq * m_q,
            )

        for h in range(N_HOP):
            origin_cw = (my_pos - 1 - h) % N_DEV
            origin_ccw = (my_pos + 1 + h) % N_DEV
            for q in range(S):
                rdma(h, q, True).wait_recv()
                rdma(h, q, False).wait_recv()
                if h + 1 < N_HOP:
                    for cw in (True, False):
                        r = rdma(h + 1, q, cw)
                        r.start()
                        sends.append(r)
                rows = pl.ds(q * m_q, m_q)
                quarter_gemm(
                    cw_ref[h, rows, :],
                    origin_cw * m_per + q * m_q,
                )
                quarter_gemm(
                    ccw_ref[h, rows, :],
                    origin_ccw * m_per + m_half + q * m_q,
                )

        for r in sends:
            r.wait_send()
        for c in copies:
            if c is not None:
                c.wait()

    return pl.pallas_call(
        body,
        out_shape=jax.ShapeDtypeStruct((N_DEV * m_per, n), jnp.bfloat16),
        in_specs=[
            pl.BlockSpec(memory_space=pltpu.MemorySpace.VMEM),
            pl.BlockSpec(memory_space=pltpu.MemorySpace.VMEM),
            pl.BlockSpec(memory_space=pl.ANY),
        ],
        out_specs=pl.BlockSpec(memory_space=pl.ANY),
        input_output_aliases={2: 0},
        scratch_shapes=[
            pltpu.VMEM((N_HOP, m_half, k), jnp.bfloat16),
            pltpu.VMEM((N_HOP, m_half, k), jnp.bfloat16),
            pltpu.VMEM((2, m_q, n), jnp.bfloat16),
            pltpu.SemaphoreType.DMA((N_HOP * S,)),
            pltpu.SemaphoreType.DMA((N_HOP * S,)),
            pltpu.SemaphoreType.DMA((N_HOP * S,)),
            pltpu.SemaphoreType.DMA((N_HOP * S,)),
            pltpu.SemaphoreType.DMA((2,)),
        ],
        compiler_params=pltpu.CompilerParams(
            collective_id=0,
            vmem_limit_bytes=128 * 1024 * 1024,
        ),
    )(A, B, jnp.zeros((N_DEV * m_per, n), jnp.bfloat16))


# device time: 248209 ns/iter; 1.0994x vs baseline; 1.0994x over previous
import jax
import jax.numpy as jnp
from jax import lax
from jax.experimental import pallas as pl
from jax.experimental.pallas import tpu as pltpu

N_DEV = 4
N_HOP = N_DEV - 1
S = 2


def kernel(A, B):
    A = A.astype(jnp.bfloat16)
    B = B.astype(jnp.bfloat16)
    m_per, k = A.shape
    n = B.shape[1]
    m_half = m_per // 2
    m_q = m_half // S

    def body(a_ref, b_ref, out_ref, cw_ref, ccw_ref, chunk_ref,
             send_cw, recv_cw, send_ccw, recv_ccw, copy_sems):
        my_pos = lax.axis_index("i")
        left = (my_pos - 1) % N_DEV
        right = (my_pos + 1) % N_DEV

        barrier_sem = pltpu.get_barrier_semaphore()
        for nbr in (left, right):
            pl.semaphore_signal(
                barrier_sem, inc=1,
                device_id=(nbr,), device_id_type=pl.DeviceIdType.MESH,
            )
        pl.semaphore_wait(barrier_sem, 2)

        def rdma(h, q, cw):
            buf, qsend, qrecv, tgt, a_off = (
                (cw_ref, send_cw, recv_cw, right, 0) if cw
                else (ccw_ref, send_ccw, recv_ccw, left, m_half)
            )
            rows = pl.ds(q * m_q, m_q)
            src = (a_ref.at[pl.ds(a_off + q * m_q, m_q), :] if h == 0
                   else buf.at[h - 1, rows, :])
            return pltpu.make_async_remote_copy(
                src_ref=src, dst_ref=buf.at[h, rows, :],
                send_sem=qsend.at[h * S + q], recv_sem=qrecv.at[h * S + q],
                device_id=(tgt,), device_id_type=pl.DeviceIdType.MESH,
            )

        copies = [None, None]
        gemm_count = [0]

        def quarter_gemm(src_view, out_row):
            j = gemm_count[0] % 2
            gemm_count[0] += 1
            if copies[j] is not None:
                copies[j].wait()
            chunk_ref[j] = jnp.dot(
                src_view, b_ref[...], preferred_element_type=jnp.float32,
            ).astype(jnp.bfloat16)
            copy = pltpu.make_async_copy(
                chunk_ref.at[j], out_ref.at[pl.ds(out_row, m_q), :],
                copy_sems.at[j],
            )
            copy.start()
            copies[j] = copy

        sends = []
        for q in range(S):
            for cw in (True, False):
                r = rdma(0, q, cw)
                r.start()
                sends.append(r)

        for qq in range(2 * S):
            quarter_gemm(
                a_ref[pl.ds(qq * m_q, m_q), :],
                my_pos * m_per + qq * m_q,
            )

        for h in range(N_HOP):
            origin_cw = (my_pos - 1 - h) % N_DEV
            origin_ccw = (my_pos + 1 + h) % N_DEV
            for q in range(S):
                rdma(h, q, True).wait_recv()
                rdma(h, q, False).wait_recv()
                if h + 1 < N_HOP:
                    for cw in (True, False):
                        r = rdma(h + 1, q, cw)
                        r.start()
                        sends.append(r)
                rows = pl.ds(q * m_q, m_q)
                quarter_gemm(
                    cw_ref[h, rows, :],
                    origin_cw * m_per + q * m_q,
                )
                quarter_gemm(
                    ccw_ref[h, rows, :],
                    origin_ccw * m_per + m_half + q * m_q,
                )

        for r in sends:
            r.wait_send()
        for c in copies:
            if c is not None:
                c.wait()

    return pl.pallas_call(
        body,
        out_shape=jax.ShapeDtypeStruct((N_DEV * m_per, n), jnp.bfloat16),
        in_specs=[
            pl.BlockSpec(memory_space=pltpu.MemorySpace.VMEM),
            pl.BlockSpec(memory_space=pltpu.MemorySpace.VMEM),
        ],
        out_specs=pl.BlockSpec(memory_space=pltpu.MemorySpace.HBM),
        scratch_shapes=[
            pltpu.VMEM((N_HOP, m_half, k), jnp.bfloat16),
            pltpu.VMEM((N_HOP, m_half, k), jnp.bfloat16),
            pltpu.VMEM((2, m_q, n), jnp.bfloat16),
            pltpu.SemaphoreType.DMA((N_HOP * S,)),
            pltpu.SemaphoreType.DMA((N_HOP * S,)),
            pltpu.SemaphoreType.DMA((N_HOP * S,)),
            pltpu.SemaphoreType.DMA((N_HOP * S,)),
            pltpu.SemaphoreType.DMA((2,)),
        ],
        compiler_params=pltpu.CompilerParams(
            collective_id=0,
            vmem_limit_bytes=128 * 1024 * 1024,
        ),
    )(A, B)
